# baseline (speedup 1.0000x reference)
import functools

import jax
import jax.numpy as jnp
from jax import lax
from jax.experimental import pallas as pl
from jax.experimental.pallas import tpu as pltpu
from jax.experimental.pallas import tpu_sc as plsc

N = 20000
E = 80000
F_IN = 16
F_EDGE = 4
B = 512
LATENT = 64

NC = 2
NS = 16
NW = NC * NS

STEP = 128
NSTEPS = 20
EPT = STEP * NSTEPS
E_PAD = NW * EPT

NPT = 640
N_PAD = NW * NPT
CNT_ROWS = 520
DEG_ROWS = N + 8
DEG_W = 16

_MESH = plsc.VectorSubcoreMesh(core_axis_name="c", subcore_axis_name="s")


def _tile_id():
    return lax.axis_index("c") * NS + lax.axis_index("s")


@functools.partial(
    pl.kernel,
    out_type=(
        jax.ShapeDtypeStruct((NC, DEG_ROWS, DEG_W), jnp.float32),
        jax.ShapeDtypeStruct((NC, CNT_ROWS, 16), jnp.float32),
    ),
    mesh=_MESH,
    compiler_params=pltpu.CompilerParams(use_tc_tiling_on_sc=False),
    scratch_types=[
        pltpu.VMEM((STEP, 16), jnp.float32),
        pltpu.VMEM((STEP, DEG_W), jnp.float32),
        pltpu.VMEM((NSTEPS, STEP), jnp.int32),
        pltpu.VMEM((5, STEP), jnp.int32),
        pltpu.VMEM_SHARED((DEG_ROWS, DEG_W), jnp.float32),
        pltpu.VMEM_SHARED((CNT_ROWS, 16), jnp.float32),
    ],
)
def _sc_prep(src_hbm, batch_hbm, zdeg_hbm, zcnt_hbm, ones_hbm, ones4_hbm,
             deg_out, cnt_out, ones_v, ones4_v, sidx_v, bidx_v, deg_sh,
             cnt_sh):
    c = lax.axis_index("c")
    s = lax.axis_index("s")
    t = _tile_id()

    @pl.when(s == 0)
    def _():
        pltpu.sync_copy(zdeg_hbm, deg_sh)
        pltpu.sync_copy(zcnt_hbm, cnt_sh)

    pltpu.sync_copy(ones_hbm, ones_v)
    pltpu.sync_copy(ones4_hbm, ones4_v)
    pltpu.sync_copy(src_hbm.at[t], sidx_v)
    pltpu.sync_copy(batch_hbm.at[t], bidx_v)
    plsc.subcore_barrier()

    for i in range(NSTEPS):
        pltpu.sync_copy(ones4_v, deg_sh.at[sidx_v.at[i]], add=True)
    for i in range(5):
        pltpu.sync_copy(ones_v, cnt_sh.at[bidx_v.at[i]], add=True)

    plsc.subcore_barrier()

    @pl.when(s == 0)
    def _():
        pltpu.sync_copy(deg_sh, deg_out.at[c])
        pltpu.sync_copy(cnt_sh, cnt_out.at[c])


_SPLAT_DNUMS = lax.GatherDimensionNumbers(
    offset_dims=(), collapsed_slice_dims=(0,), start_index_map=(0,))


def _splat_lane(vec, k):
    return lax.gather(vec, jnp.full((16, 1), k, jnp.int32), _SPLAT_DNUMS,
                      (1,), mode=lax.GatherScatterMode.PROMISE_IN_BOUNDS)


def _make_edge_kernel(out_ch, nsteps):
    roww = F_EDGE * out_ch
    mesh1 = plsc.VectorSubcoreMesh(
        core_axis_name="c", subcore_axis_name="s", num_cores=1)

    @functools.partial(
        pl.kernel,
        out_type=jax.ShapeDtypeStruct((N, out_ch), jnp.float32),
        mesh=mesh1,
        compiler_params=pltpu.CompilerParams(use_tc_tiling_on_sc=False),
        scratch_types=[
            pltpu.VMEM((nsteps, STEP), jnp.int32),
            pltpu.VMEM((nsteps, STEP), jnp.int32),
            pltpu.VMEM((2, STEP, 16), jnp.float32),
            pltpu.VMEM((2, STEP, roww), jnp.float32),
            pltpu.VMEM((STEP, out_ch), jnp.float32),
            pltpu.VMEM_SHARED((N, out_ch), jnp.float32),
            pltpu.SemaphoreType.DMA,
            pltpu.SemaphoreType.DMA,
            pltpu.SemaphoreType.DMA,
            pltpu.SemaphoreType.DMA,
        ],
    )
    def edge_kernel(pn_hbm, srcg_hbm, dst_hbm, ea_hbm, zero_hbm, agg_out,
                    sidx_v, didx_v, ea_v, rows_v, msg_v, agg_sh,
                    gsem0, gsem1, esem0, esem1):
        s = lax.axis_index("s")

        @pl.when(s == 0)
        def _():
            pltpu.sync_copy(zero_hbm, agg_sh)

        pltpu.sync_copy(srcg_hbm.at[s], sidx_v)
        pltpu.sync_copy(dst_hbm.at[s], didx_v)
        plsc.subcore_barrier()

        gsems = (gsem0, gsem1)
        esems = (esem0, esem1)
        gcps = [None, None]
        ecps = [None, None]
        gcps[0] = pltpu.async_copy(pn_hbm.at[sidx_v.at[0]], rows_v.at[0],
                                   gsems[0])
        ecps[0] = pltpu.async_copy(ea_hbm.at[s, 0], ea_v.at[0], esems[0])
        for step in range(nsteps):
            buf = step % 2
            if step + 1 < nsteps:
                nbuf = (step + 1) % 2
                gcps[nbuf] = pltpu.async_copy(
                    pn_hbm.at[sidx_v.at[step + 1]], rows_v.at[nbuf],
                    gsems[nbuf])
                ecps[nbuf] = pltpu.async_copy(
                    ea_hbm.at[s, step + 1], ea_v.at[nbuf], esems[nbuf])
            gcps[buf].wait()
            ecps[buf].wait()

            def edge_body(e, carry):
                eav = ea_v[buf, e]
                eak = [_splat_lane(eav, k) for k in range(F_EDGE)]
                for j in range(out_ch // 16):
                    acc = eak[0] * rows_v[buf, e, pl.ds(j * 16, 16)]
                    for k in range(1, F_EDGE):
                        acc = acc + eak[k] * rows_v[
                            buf, e, pl.ds(k * out_ch + j * 16, 16)]
                    msg_v[e, pl.ds(j * 16, 16)] = acc
                return carry

            lax.fori_loop(0, STEP, edge_body, 0)
            pltpu.sync_copy(msg_v, agg_sh.at[didx_v.at[step]], add=True)

        plsc.subcore_barrier()

        @pl.when(s == 0)
        def _():
            pltpu.sync_copy(agg_sh, agg_out)

    return edge_kernel


_edge32_half = _make_edge_kernel(32, NSTEPS)
_edge32_full = _make_edge_kernel(32, 2 * NSTEPS)


@functools.partial(
    pl.kernel,
    out_type=jax.ShapeDtypeStruct((NC, CNT_ROWS, 64), jnp.float32),
    mesh=_MESH,
    compiler_params=pltpu.CompilerParams(use_tc_tiling_on_sc=False),
    scratch_types=[
        pltpu.VMEM((5, STEP), jnp.int32),
        pltpu.VMEM((STEP, 64), jnp.float32),
        pltpu.VMEM_SHARED((CNT_ROWS, 64), jnp.float32),
    ],
)
def _sc_pool(h_hbm, batch_hbm, zpool_hbm, pool_out, bidx_v, rows_v, pool_sh):
    c = lax.axis_index("c")
    s = lax.axis_index("s")
    t = _tile_id()

    @pl.when(s == 0)
    def _():
        pltpu.sync_copy(zpool_hbm, pool_sh)

    pltpu.sync_copy(batch_hbm.at[t], bidx_v)
    plsc.subcore_barrier()

    for i in range(5):
        pltpu.sync_copy(h_hbm.at[pl.ds(t * NPT + i * STEP, STEP)], rows_v)
        pltpu.sync_copy(rows_v, pool_sh.at[bidx_v.at[i]], add=True)

    plsc.subcore_barrier()

    @pl.when(s == 0)
    def _():
        pltpu.sync_copy(pool_sh, pool_out.at[c])


RBLK = 2000
NGRID = N // RBLK


def _norm_blk(dA_blk, dB_blk):
    deg = dA_blk[:, 0:1] + dB_blk[:, 0:1]
    return jnp.where(deg > 0, 1.0 / deg, 0.0)


def _rspec(cols):
    return pl.BlockSpec((RBLK, cols), lambda b: (b, 0))


def _full(shape):
    return pl.BlockSpec(shape, lambda b: tuple(0 for _ in shape))


def _tc_prep1(x, dA, dB, wcat1, wres1t):
    def body(x_ref, dA_ref, dB_ref, wc_ref, wr_ref, pn_ref, res_ref):
        norm = _norm_blk(dA_ref[...], dB_ref[...])
        xn = x_ref[...] * norm
        pn_ref[...] = jnp.dot(xn, wc_ref[...],
                              preferred_element_type=jnp.float32)
        res_ref[...] = jnp.dot(x_ref[...], wr_ref[...],
                               preferred_element_type=jnp.float32)

    return pl.pallas_call(
        body,
        grid=(NGRID,),
        in_specs=[_rspec(F_IN), _rspec(DEG_W), _rspec(DEG_W),
                  _full((F_IN, F_EDGE * 32)), _full((F_IN, 32))],
        out_specs=(_rspec(F_EDGE * 32), _rspec(32)),
        out_shape=(jax.ShapeDtypeStruct((N, F_EDGE * 32), jnp.float32),
                   jax.ShapeDtypeStruct((N, 32), jnp.float32)),
    )(x, dA, dB, wcat1, wres1t)


def _accum_stats(t, b, ssum_ref, ssq_ref):
    s = jnp.sum(t, axis=0, keepdims=True)
    q = jnp.sum(t * t, axis=0, keepdims=True)

    @pl.when(b == 0)
    def _():
        ssum_ref[...] = s
        ssq_ref[...] = q

    @pl.when(b > 0)
    def _():
        ssum_ref[...] += s
        ssq_ref[...] += q


def _tc_stats1(aggA, aggB, res1, b1):
    def body(aggA_ref, aggB_ref, res_ref, b_ref, t_ref, ssum_ref, ssq_ref):
        t = aggA_ref[...] + aggB_ref[...] + res_ref[...] + b_ref[...]
        t_ref[...] = t
        _accum_stats(t, pl.program_id(0), ssum_ref, ssq_ref)

    st_spec = pl.BlockSpec((1, 32), lambda b: (0, 0))
    return pl.pallas_call(
        body,
        grid=(NGRID,),
        in_specs=[_rspec(32), _rspec(32), _rspec(32), _full((1, 32))],
        out_specs=(_rspec(32), st_spec, st_spec),
        out_shape=(jax.ShapeDtypeStruct((N, 32), jnp.float32),
                   jax.ShapeDtypeStruct((1, 32), jnp.float32),
                   jax.ShapeDtypeStruct((1, 32), jnp.float32)),
    )(aggA, aggB, res1, b1)


def _tc_stats2(agg2a, agg2b, res2, b2):
    def body(agga_ref, aggb_ref, res_ref, b_ref, t_ref, ssum_ref, ssq_ref):
        agg = jnp.concatenate([agga_ref[...], aggb_ref[...]], axis=1)
        t = agg + res_ref[...] + b_ref[...]
        t_ref[...] = t
        _accum_stats(t, pl.program_id(0), ssum_ref, ssq_ref)

    st_spec = pl.BlockSpec((1, 64), lambda b: (0, 0))
    return pl.pallas_call(
        body,
        grid=(NGRID,),
        in_specs=[_rspec(32), _rspec(32), _rspec(64), _full((1, 64))],
        out_specs=(_rspec(64), st_spec, st_spec),
        out_shape=(jax.ShapeDtypeStruct((N, 64), jnp.float32),
                   jax.ShapeDtypeStruct((1, 64), jnp.float32),
                   jax.ShapeDtypeStruct((1, 64), jnp.float32)),
    )(agg2a, agg2b, res2, b2)


def _bn_relu_from_stats(t, ssum, ssq, g, be):
    mu = ssum * (1.0 / N)
    var = ssq * (1.0 / N) - mu * mu
    return jnp.maximum(g * (t - mu) / jnp.sqrt(var + 1e-5) + be, 0.0)


def _tc_apply1(t1, ssum1, ssq1, g1, be1, dA, dB, wcat2a, wcat2b, wres2t):
    def body(t_ref, ssum_ref, ssq_ref, g_ref, be_ref, dA_ref, dB_ref,
             wca_ref, wcb_ref, wr_ref, pna_ref, pnb_ref, res2_ref):
        h = _bn_relu_from_stats(t_ref[...], ssum_ref[...], ssq_ref[...],
                                g_ref[...], be_ref[...])
        norm = _norm_blk(dA_ref[...], dB_ref[...])
        hn = h * norm
        pna_ref[...] = jnp.dot(hn, wca_ref[...],
                               preferred_element_type=jnp.float32)
        pnb_ref[...] = jnp.dot(hn, wcb_ref[...],
                               preferred_element_type=jnp.float32)
        res2_ref[...] = jnp.dot(h, wr_ref[...],
                                preferred_element_type=jnp.float32)

    return pl.pallas_call(
        body,
        grid=(NGRID,),
        in_specs=[_rspec(32), _full((1, 32)), _full((1, 32)),
                  _full((1, 32)), _full((1, 32)),
                  _rspec(DEG_W), _rspec(DEG_W),
                  _full((32, F_EDGE * 32)), _full((32, F_EDGE * 32)),
                  _full((32, 64))],
        out_specs=(_rspec(F_EDGE * 32), _rspec(F_EDGE * 32), _rspec(64)),
        out_shape=(jax.ShapeDtypeStruct((N, F_EDGE * 32), jnp.float32),
                   jax.ShapeDtypeStruct((N, F_EDGE * 32), jnp.float32),
                   jax.ShapeDtypeStruct((N, 64), jnp.float32)),
    )(t1, ssum1, ssq1, g1, be1, dA, dB, wcat2a, wcat2b, wres2t)


PBLK = 2048
NPGRID = N_PAD // PBLK


def _tc_apply2(t2, ssum2, ssq2, g2, be2):
    def body(t_ref, ssum_ref, ssq_ref, g_ref, be_ref, h_ref):
        h = _bn_relu_from_stats(t_ref[...], ssum_ref[...], ssq_ref[...],
                                g_ref[...], be_ref[...])
        row0 = pl.program_id(0) * PBLK
        rows = row0 + lax.broadcasted_iota(jnp.int32, (PBLK, 64), 0)
        h_ref[...] = jnp.where(rows < N, h, 0.0)

    return pl.pallas_call(
        body,
        grid=(NPGRID,),
        in_specs=[pl.BlockSpec((PBLK, 64), lambda b: (b, 0)),
                  _full((1, 64)), _full((1, 64)), _full((1, 64)),
                  _full((1, 64))],
        out_specs=pl.BlockSpec((PBLK, 64), lambda b: (b, 0)),
        out_shape=jax.ShapeDtypeStruct((N_PAD, 64), jnp.float32),
    )(t2, ssum2, ssq2, g2, be2)


def _tc_head(pools, cA, cB, wfc1t, bfc1, wfc2t, bfc2):
    def body(pool_ref, cA_ref, cB_ref, w1_ref, b1_ref, w2_ref, b2_ref,
             mu_ref, ls_ref):
        sums = pool_ref[0, 0:B, :] + pool_ref[1, 0:B, :]
        cnt = cA_ref[0:B, 0:1] + cB_ref[0:B, 0:1]
        pooled = sums / jnp.maximum(cnt, 1.0)
        z1 = jnp.maximum(
            jnp.dot(pooled, w1_ref[...], preferred_element_type=jnp.float32)
            + b1_ref[...], 0.0)
        z = jnp.dot(z1, w2_ref[...],
                    preferred_element_type=jnp.float32) + b2_ref[...]
        mu_ref[...] = z[:, 0:LATENT]
        ls_ref[...] = z[:, LATENT:2 * LATENT]

    return pl.pallas_call(
        body,
        out_shape=(jax.ShapeDtypeStruct((B, LATENT), jnp.float32),
                   jax.ShapeDtypeStruct((B, LATENT), jnp.float32)),
    )(pools, cA, cB, wfc1t, bfc1, wfc2t, bfc2)


def _wcat(w_edge, out_ch, in_ch):
    return (w_edge.reshape(out_ch, in_ch, F_EDGE)
            .transpose(1, 2, 0).reshape(in_ch, F_EDGE * out_ch))


def kernel(x, edge_index, edge_attr, batch, W_edge1, b1, W_res1, g1, be1,
           W_edge2, b2, W_res2, g2, be2, W_fc1, b_fc1, W_fc2, b_fc2):
    src = edge_index[0]
    dst = edge_index[1]
    epad = E_PAD - E
    half = E_PAD // 2
    srcg = jnp.concatenate([src, jnp.zeros((epad,), jnp.int32)])
    srcd = jnp.concatenate([src, jnp.full((epad,), N, jnp.int32)])
    srcd = srcd.reshape(NW, NSTEPS, STEP)
    dstp = jnp.concatenate([dst, jnp.zeros((epad,), jnp.int32)])
    eap = jnp.pad(edge_attr, ((0, epad), (0, 16 - F_EDGE)))

    srcgA = srcg[:half].reshape(NS, NSTEPS, STEP)
    srcgB = srcg[half:].reshape(NS, NSTEPS, STEP)
    dstA = dstp[:half].reshape(NS, NSTEPS, STEP)
    dstB = dstp[half:].reshape(NS, NSTEPS, STEP)
    eaA = eap[:half].reshape(NS, NSTEPS, STEP, 16)
    eaB = eap[half:].reshape(NS, NSTEPS, STEP, 16)
    srcg_all = srcg.reshape(NS, 2 * NSTEPS, STEP)
    dst_all = dstp.reshape(NS, 2 * NSTEPS, STEP)
    ea_all = eap.reshape(NS, 2 * NSTEPS, STEP, 16)
    batchp = jnp.concatenate(
        [batch, jnp.full((N_PAD - N,), B, jnp.int32)])
    batchp = batchp.reshape(NW, 5, STEP)

    zdeg = jnp.zeros((DEG_ROWS, DEG_W), jnp.float32)
    zcnt = jnp.zeros((CNT_ROWS, 16), jnp.float32)
    z32 = jnp.zeros((N, 32), jnp.float32)
    zpool = jnp.zeros((CNT_ROWS, 64), jnp.float32)
    ones16 = jnp.ones((STEP, 16), jnp.float32)
    ones4 = jnp.ones((STEP, DEG_W), jnp.float32)

    wcat1 = _wcat(W_edge1, 32, F_IN)
    wcat2 = _wcat(W_edge2, 64, 32).reshape(32, F_EDGE, 64)
    wcat2a = wcat2[:, :, :32].reshape(32, F_EDGE * 32)
    wcat2b = wcat2[:, :, 32:].reshape(32, F_EDGE * 32)

    degs, cnts = _sc_prep(srcd, batchp, zdeg, zcnt, ones16, ones4)
    dA, dB = degs[0], degs[1]

    pn1, res1 = _tc_prep1(x, dA, dB, wcat1, W_res1.T)
    aggA = _edge32_half(pn1, srcgA, dstA, eaA, z32)
    aggB = _edge32_half(pn1, srcgB, dstB, eaB, z32)
    t1, ssum1, ssq1 = _tc_stats1(aggA, aggB, res1, b1.reshape(1, 32))
    pn2a, pn2b, res2 = _tc_apply1(t1, ssum1, ssq1, g1.reshape(1, 32),
                                  be1.reshape(1, 32), dA, dB,
                                  wcat2a, wcat2b, W_res2.T)
    agg2a = _edge32_full(pn2a, srcg_all, dst_all, ea_all, z32)
    agg2b = _edge32_full(pn2b, srcg_all, dst_all, ea_all, z32)
    t2, ssum2, ssq2 = _tc_stats2(agg2a, agg2b, res2, b2.reshape(1, 64))
    h2 = _tc_apply2(t2, ssum2, ssq2, g2.reshape(1, 64), be2.reshape(1, 64))
    pools = _sc_pool(h2, batchp, zpool)
    mu, log_sigma = _tc_head(pools, cnts[0], cnts[1], W_fc1.T,
                             b_fc1.reshape(1, -1), W_fc2.T,
                             b_fc2.reshape(1, -1))
    return (mu, log_sigma)


# speedup vs baseline: 1.0921x; 1.0921x over previous
import functools

import jax
import jax.numpy as jnp
from jax import lax
from jax.experimental import pallas as pl
from jax.experimental.pallas import tpu as pltpu
from jax.experimental.pallas import tpu_sc as plsc

N = 20000
E = 80000
F_IN = 16
F_EDGE = 4
B = 512
LATENT = 64

NC = 2
NS = 16
NW = NC * NS

STEP = 128
NSTEPS = 20
EPT = STEP * NSTEPS
E_PAD = NW * EPT

NPT = 640
N_PAD = NW * NPT
CNT_ROWS = 520
DEG_ROWS = N + 8
DEG_W = 16

_MESH = plsc.VectorSubcoreMesh(core_axis_name="c", subcore_axis_name="s")


def _tile_id():
    return lax.axis_index("c") * NS + lax.axis_index("s")


@functools.partial(
    pl.kernel,
    out_type=(
        jax.ShapeDtypeStruct((NC, DEG_ROWS, DEG_W), jnp.float32),
        jax.ShapeDtypeStruct((NC, CNT_ROWS, 16), jnp.float32),
    ),
    mesh=_MESH,
    compiler_params=pltpu.CompilerParams(use_tc_tiling_on_sc=False),
    scratch_types=[
        pltpu.VMEM((STEP, 16), jnp.float32),
        pltpu.VMEM((STEP, DEG_W), jnp.float32),
        pltpu.VMEM((NSTEPS, STEP), jnp.int32),
        pltpu.VMEM((5, STEP), jnp.int32),
        pltpu.VMEM_SHARED((DEG_ROWS, DEG_W), jnp.float32),
        pltpu.VMEM_SHARED((CNT_ROWS, 16), jnp.float32),
    ],
)
def _sc_prep(src_hbm, batch_hbm, zdeg_hbm, zcnt_hbm, ones_hbm, ones4_hbm,
             deg_out, cnt_out, ones_v, ones4_v, sidx_v, bidx_v, deg_sh,
             cnt_sh):
    c = lax.axis_index("c")
    s = lax.axis_index("s")
    t = _tile_id()

    @pl.when(s == 0)
    def _():
        pltpu.sync_copy(zdeg_hbm, deg_sh)
        pltpu.sync_copy(zcnt_hbm, cnt_sh)

    pltpu.sync_copy(ones_hbm, ones_v)
    pltpu.sync_copy(ones4_hbm, ones4_v)
    pltpu.sync_copy(src_hbm.at[t], sidx_v)
    pltpu.sync_copy(batch_hbm.at[t], bidx_v)
    plsc.subcore_barrier()

    for i in range(NSTEPS):
        pltpu.sync_copy(ones4_v, deg_sh.at[sidx_v.at[i]], add=True)
    for i in range(5):
        pltpu.sync_copy(ones_v, cnt_sh.at[bidx_v.at[i]], add=True)

    plsc.subcore_barrier()

    @pl.when(s == 0)
    def _():
        pltpu.sync_copy(deg_sh, deg_out.at[c])
        pltpu.sync_copy(cnt_sh, cnt_out.at[c])


_SPLAT_DNUMS = lax.GatherDimensionNumbers(
    offset_dims=(), collapsed_slice_dims=(0,), start_index_map=(0,))


def _splat_lane(vec, k):
    return lax.gather(vec, jnp.full((16, 1), k, jnp.int32), _SPLAT_DNUMS,
                      (1,), mode=lax.GatherScatterMode.PROMISE_IN_BOUNDS)


def _make_edge_kernel(out_ch):
    roww = F_EDGE * out_ch

    @functools.partial(
        pl.kernel,
        out_type=jax.ShapeDtypeStruct((NC, N, out_ch), jnp.float32),
        mesh=_MESH,
        compiler_params=pltpu.CompilerParams(use_tc_tiling_on_sc=False),
        scratch_types=[
            pltpu.VMEM((NSTEPS, STEP), jnp.int32),
            pltpu.VMEM((NSTEPS, STEP), jnp.int32),
            pltpu.VMEM((EPT, 16), jnp.float32),
            pltpu.VMEM((2, STEP, roww), jnp.float32),
            pltpu.VMEM((STEP, out_ch), jnp.float32),
            pltpu.VMEM_SHARED((N, out_ch), jnp.float32),
            pltpu.SemaphoreType.DMA,
            pltpu.SemaphoreType.DMA,
        ],
    )
    def edge_kernel(pn_hbm, srcg_hbm, dst_hbm, ea_hbm, zero_hbm, agg_out,
                    sidx_v, didx_v, ea_v, rows_v, msg_v, agg_sh, sem0, sem1):
        c = lax.axis_index("c")
        s = lax.axis_index("s")
        t = _tile_id()

        @pl.when(s == 0)
        def _():
            pltpu.sync_copy(zero_hbm, agg_sh)

        pltpu.sync_copy(srcg_hbm.at[t], sidx_v)
        pltpu.sync_copy(dst_hbm.at[t], didx_v)
        pltpu.sync_copy(ea_hbm.at[t], ea_v)
        plsc.subcore_barrier()

        sems = (sem0, sem1)
        cps = [None, None]
        cps[0] = pltpu.async_copy(pn_hbm.at[sidx_v.at[0]], rows_v.at[0],
                                  sems[0])
        for step in range(NSTEPS):
            buf = step % 2
            if step + 1 < NSTEPS:
                nbuf = (step + 1) % 2
                cps[nbuf] = pltpu.async_copy(
                    pn_hbm.at[sidx_v.at[step + 1]], rows_v.at[nbuf],
                    sems[nbuf])
            cps[buf].wait()

            def edge_body(e, carry):
                eav = ea_v[step * STEP + e]
                eak = [_splat_lane(eav, k) for k in range(F_EDGE)]
                for j in range(out_ch // 16):
                    acc = eak[0] * rows_v[buf, e, pl.ds(j * 16, 16)]
                    for k in range(1, F_EDGE):
                        acc = acc + eak[k] * rows_v[
                            buf, e, pl.ds(k * out_ch + j * 16, 16)]
                    msg_v[e, pl.ds(j * 16, 16)] = acc
                return carry

            lax.fori_loop(0, STEP, edge_body, 0, unroll=2)
            pltpu.sync_copy(msg_v, agg_sh.at[didx_v.at[step]], add=True)

        plsc.subcore_barrier()

        @pl.when(s == 0)
        def _():
            pltpu.sync_copy(agg_sh, agg_out.at[c])

    return edge_kernel


_edge32 = _make_edge_kernel(32)


@functools.partial(
    pl.kernel,
    out_type=jax.ShapeDtypeStruct((NC, CNT_ROWS, 64), jnp.float32),
    mesh=_MESH,
    compiler_params=pltpu.CompilerParams(use_tc_tiling_on_sc=False),
    scratch_types=[
        pltpu.VMEM((5, STEP), jnp.int32),
        pltpu.VMEM((STEP, 64), jnp.float32),
        pltpu.VMEM_SHARED((CNT_ROWS, 64), jnp.float32),
    ],
)
def _sc_pool(h_hbm, batch_hbm, zpool_hbm, pool_out, bidx_v, rows_v, pool_sh):
    c = lax.axis_index("c")
    s = lax.axis_index("s")
    t = _tile_id()

    @pl.when(s == 0)
    def _():
        pltpu.sync_copy(zpool_hbm, pool_sh)

    pltpu.sync_copy(batch_hbm.at[t], bidx_v)
    plsc.subcore_barrier()

    for i in range(5):
        pltpu.sync_copy(h_hbm.at[pl.ds(t * NPT + i * STEP, STEP)], rows_v)
        pltpu.sync_copy(rows_v, pool_sh.at[bidx_v.at[i]], add=True)

    plsc.subcore_barrier()

    @pl.when(s == 0)
    def _():
        pltpu.sync_copy(pool_sh, pool_out.at[c])


RBLK = 2000
NGRID = N // RBLK


def _norm_blk(dA_blk, dB_blk):
    deg = dA_blk[:, 0:1] + dB_blk[:, 0:1]
    return jnp.where(deg > 0, 1.0 / deg, 0.0)


def _rspec(cols):
    return pl.BlockSpec((RBLK, cols), lambda b: (b, 0))


def _full(shape):
    return pl.BlockSpec(shape, lambda b: tuple(0 for _ in shape))


def _tc_prep1(x, dA, dB, wcat1, wres1t):
    def body(x_ref, dA_ref, dB_ref, wc_ref, wr_ref, pn_ref, res_ref):
        norm = _norm_blk(dA_ref[...], dB_ref[...])
        xn = x_ref[...] * norm
        pn_ref[...] = jnp.dot(xn, wc_ref[...],
                              preferred_element_type=jnp.float32)
        res_ref[...] = jnp.dot(x_ref[...], wr_ref[...],
                               preferred_element_type=jnp.float32)

    return pl.pallas_call(
        body,
        grid=(NGRID,),
        in_specs=[_rspec(F_IN), _rspec(DEG_W), _rspec(DEG_W),
                  _full((F_IN, F_EDGE * 32)), _full((F_IN, 32))],
        out_specs=(_rspec(F_EDGE * 32), _rspec(32)),
        out_shape=(jax.ShapeDtypeStruct((N, F_EDGE * 32), jnp.float32),
                   jax.ShapeDtypeStruct((N, 32), jnp.float32)),
    )(x, dA, dB, wcat1, wres1t)


def _accum_stats(t, b, ssum_ref, ssq_ref):
    s = jnp.sum(t, axis=0, keepdims=True)
    q = jnp.sum(t * t, axis=0, keepdims=True)

    @pl.when(b == 0)
    def _():
        ssum_ref[...] = s
        ssq_ref[...] = q

    @pl.when(b > 0)
    def _():
        ssum_ref[...] += s
        ssq_ref[...] += q


def _tc_stats1(aggs1, res1, b1):
    def body(agg_ref, res_ref, b_ref, t_ref, ssum_ref, ssq_ref):
        t = agg_ref[0] + agg_ref[1] + res_ref[...] + b_ref[...]
        t_ref[...] = t
        _accum_stats(t, pl.program_id(0), ssum_ref, ssq_ref)

    st_spec = pl.BlockSpec((1, 32), lambda b: (0, 0))
    return pl.pallas_call(
        body,
        grid=(NGRID,),
        in_specs=[pl.BlockSpec((2, RBLK, 32), lambda b: (0, b, 0)),
                  _rspec(32), _full((1, 32))],
        out_specs=(_rspec(32), st_spec, st_spec),
        out_shape=(jax.ShapeDtypeStruct((N, 32), jnp.float32),
                   jax.ShapeDtypeStruct((1, 32), jnp.float32),
                   jax.ShapeDtypeStruct((1, 32), jnp.float32)),
    )(aggs1, res1, b1)


def _tc_stats2(aggs2a, aggs2b, res2, b2):
    def body(agga_ref, aggb_ref, res_ref, b_ref, t_ref, ssum_ref, ssq_ref):
        agg = jnp.concatenate(
            [agga_ref[0] + agga_ref[1], aggb_ref[0] + aggb_ref[1]], axis=1)
        t = agg + res_ref[...] + b_ref[...]
        t_ref[...] = t
        _accum_stats(t, pl.program_id(0), ssum_ref, ssq_ref)

    st_spec = pl.BlockSpec((1, 64), lambda b: (0, 0))
    return pl.pallas_call(
        body,
        grid=(NGRID,),
        in_specs=[pl.BlockSpec((2, RBLK, 32), lambda b: (0, b, 0)),
                  pl.BlockSpec((2, RBLK, 32), lambda b: (0, b, 0)),
                  _rspec(64), _full((1, 64))],
        out_specs=(_rspec(64), st_spec, st_spec),
        out_shape=(jax.ShapeDtypeStruct((N, 64), jnp.float32),
                   jax.ShapeDtypeStruct((1, 64), jnp.float32),
                   jax.ShapeDtypeStruct((1, 64), jnp.float32)),
    )(aggs2a, aggs2b, res2, b2)


def _bn_relu_from_stats(t, ssum, ssq, g, be):
    mu = ssum * (1.0 / N)
    var = ssq * (1.0 / N) - mu * mu
    return jnp.maximum(g * (t - mu) / jnp.sqrt(var + 1e-5) + be, 0.0)


def _tc_apply1(t1, ssum1, ssq1, g1, be1, dA, dB, wcat2a, wcat2b, wres2t):
    def body(t_ref, ssum_ref, ssq_ref, g_ref, be_ref, dA_ref, dB_ref,
             wca_ref, wcb_ref, wr_ref, pna_ref, pnb_ref, res2_ref):
        h = _bn_relu_from_stats(t_ref[...], ssum_ref[...], ssq_ref[...],
                                g_ref[...], be_ref[...])
        norm = _norm_blk(dA_ref[...], dB_ref[...])
        hn = h * norm
        pna_ref[...] = jnp.dot(hn, wca_ref[...],
                               preferred_element_type=jnp.float32)
        pnb_ref[...] = jnp.dot(hn, wcb_ref[...],
                               preferred_element_type=jnp.float32)
        res2_ref[...] = jnp.dot(h, wr_ref[...],
                                preferred_element_type=jnp.float32)

    return pl.pallas_call(
        body,
        grid=(NGRID,),
        in_specs=[_rspec(32), _full((1, 32)), _full((1, 32)),
                  _full((1, 32)), _full((1, 32)),
                  _rspec(DEG_W), _rspec(DEG_W),
                  _full((32, F_EDGE * 32)), _full((32, F_EDGE * 32)),
                  _full((32, 64))],
        out_specs=(_rspec(F_EDGE * 32), _rspec(F_EDGE * 32), _rspec(64)),
        out_shape=(jax.ShapeDtypeStruct((N, F_EDGE * 32), jnp.float32),
                   jax.ShapeDtypeStruct((N, F_EDGE * 32), jnp.float32),
                   jax.ShapeDtypeStruct((N, 64), jnp.float32)),
    )(t1, ssum1, ssq1, g1, be1, dA, dB, wcat2a, wcat2b, wres2t)


PBLK = 2048
NPGRID = N_PAD // PBLK


def _tc_apply2(t2, ssum2, ssq2, g2, be2):
    def body(t_ref, ssum_ref, ssq_ref, g_ref, be_ref, h_ref):
        h = _bn_relu_from_stats(t_ref[...], ssum_ref[...], ssq_ref[...],
                                g_ref[...], be_ref[...])
        row0 = pl.program_id(0) * PBLK
        rows = row0 + lax.broadcasted_iota(jnp.int32, (PBLK, 64), 0)
        h_ref[...] = jnp.where(rows < N, h, 0.0)

    return pl.pallas_call(
        body,
        grid=(NPGRID,),
        in_specs=[pl.BlockSpec((PBLK, 64), lambda b: (b, 0)),
                  _full((1, 64)), _full((1, 64)), _full((1, 64)),
                  _full((1, 64))],
        out_specs=pl.BlockSpec((PBLK, 64), lambda b: (b, 0)),
        out_shape=jax.ShapeDtypeStruct((N_PAD, 64), jnp.float32),
    )(t2, ssum2, ssq2, g2, be2)


def _tc_head(pools, cA, cB, wfc1t, bfc1, wfc2t, bfc2):
    def body(pool_ref, cA_ref, cB_ref, w1_ref, b1_ref, w2_ref, b2_ref,
             mu_ref, ls_ref):
        sums = pool_ref[0, 0:B, :] + pool_ref[1, 0:B, :]
        cnt = cA_ref[0:B, 0:1] + cB_ref[0:B, 0:1]
        pooled = sums / jnp.maximum(cnt, 1.0)
        z1 = jnp.maximum(
            jnp.dot(pooled, w1_ref[...], preferred_element_type=jnp.float32)
            + b1_ref[...], 0.0)
        z = jnp.dot(z1, w2_ref[...],
                    preferred_element_type=jnp.float32) + b2_ref[...]
        mu_ref[...] = z[:, 0:LATENT]
        ls_ref[...] = z[:, LATENT:2 * LATENT]

    return pl.pallas_call(
        body,
        out_shape=(jax.ShapeDtypeStruct((B, LATENT), jnp.float32),
                   jax.ShapeDtypeStruct((B, LATENT), jnp.float32)),
    )(pools, cA, cB, wfc1t, bfc1, wfc2t, bfc2)


def _wcat(w_edge, out_ch, in_ch):
    return (w_edge.reshape(out_ch, in_ch, F_EDGE)
            .transpose(1, 2, 0).reshape(in_ch, F_EDGE * out_ch))


def kernel(x, edge_index, edge_attr, batch, W_edge1, b1, W_res1, g1, be1,
           W_edge2, b2, W_res2, g2, be2, W_fc1, b_fc1, W_fc2, b_fc2):
    src = edge_index[0]
    dst = edge_index[1]
    epad = E_PAD - E
    srcg = jnp.concatenate([src, jnp.zeros((epad,), jnp.int32)])
    srcg = srcg.reshape(NW, NSTEPS, STEP)
    srcd = jnp.concatenate([src, jnp.full((epad,), N, jnp.int32)])
    srcd = srcd.reshape(NW, NSTEPS, STEP)
    dstp = jnp.concatenate([dst, jnp.zeros((epad,), jnp.int32)])
    dstp = dstp.reshape(NW, NSTEPS, STEP)
    eap = jnp.pad(edge_attr, ((0, epad), (0, 16 - F_EDGE)))
    eap = eap.reshape(NW, EPT, 16)
    batchp = jnp.concatenate(
        [batch, jnp.full((N_PAD - N,), B, jnp.int32)])
    batchp = batchp.reshape(NW, 5, STEP)

    zdeg = jnp.zeros((DEG_ROWS, DEG_W), jnp.float32)
    zcnt = jnp.zeros((CNT_ROWS, 16), jnp.float32)
    z32 = jnp.zeros((N, 32), jnp.float32)
    zpool = jnp.zeros((CNT_ROWS, 64), jnp.float32)
    ones16 = jnp.ones((STEP, 16), jnp.float32)
    ones4 = jnp.ones((STEP, DEG_W), jnp.float32)

    wcat1 = _wcat(W_edge1, 32, F_IN)
    wcat2 = _wcat(W_edge2, 64, 32).reshape(32, F_EDGE, 64)
    wcat2a = wcat2[:, :, :32].reshape(32, F_EDGE * 32)
    wcat2b = wcat2[:, :, 32:].reshape(32, F_EDGE * 32)

    degs, cnts = _sc_prep(srcd, batchp, zdeg, zcnt, ones16, ones4)
    dA, dB = degs[0], degs[1]

    pn1, res1 = _tc_prep1(x, dA, dB, wcat1, W_res1.T)
    aggs1 = _edge32(pn1, srcg, dstp, eap, z32)
    t1, ssum1, ssq1 = _tc_stats1(aggs1, res1, b1.reshape(1, 32))
    pn2a, pn2b, res2 = _tc_apply1(t1, ssum1, ssq1, g1.reshape(1, 32),
                                  be1.reshape(1, 32), dA, dB,
                                  wcat2a, wcat2b, W_res2.T)
    aggs2a = _edge32(pn2a, srcg, dstp, eap, z32)
    aggs2b = _edge32(pn2b, srcg, dstp, eap, z32)
    t2, ssum2, ssq2 = _tc_stats2(aggs2a, aggs2b, res2, b2.reshape(1, 64))
    h2 = _tc_apply2(t2, ssum2, ssq2, g2.reshape(1, 64), be2.reshape(1, 64))
    pools = _sc_pool(h2, batchp, zpool)
    mu, log_sigma = _tc_head(pools, cnts[0], cnts[1], W_fc1.T,
                             b_fc1.reshape(1, -1), W_fc2.T,
                             b_fc2.reshape(1, -1))
    return (mu, log_sigma)


# speedup vs baseline: 1.5912x; 1.4570x over previous
import functools

import jax
import jax.numpy as jnp
from jax import lax
from jax.experimental import pallas as pl
from jax.experimental.pallas import tpu as pltpu
from jax.experimental.pallas import tpu_sc as plsc

N = 20000
E = 80000
F_IN = 16
F_EDGE = 4
B = 512
LATENT = 64

NC = 2
NS = 16
NW = NC * NS

STEP = 128
NSTEPS = 20
EPT = STEP * NSTEPS
E_PAD = NW * EPT

NPT = 640
N_PAD = NW * NPT
CNT_ROWS = 640
DEG_ROWS = N_PAD
DEG_W = 16

_MESH = plsc.VectorSubcoreMesh(core_axis_name="c", subcore_axis_name="s")


def _tile_id():
    return lax.axis_index("c") * NS + lax.axis_index("s")


@functools.partial(
    pl.kernel,
    out_type=(
        jax.ShapeDtypeStruct((NC, DEG_ROWS, DEG_W), jnp.float32),
        jax.ShapeDtypeStruct((NC, CNT_ROWS, 16), jnp.float32),
    ),
    mesh=_MESH,
    compiler_params=pltpu.CompilerParams(use_tc_tiling_on_sc=False),
    scratch_types=[
        pltpu.VMEM((STEP, 16), jnp.float32),
        pltpu.VMEM((STEP, DEG_W), jnp.float32),
        pltpu.VMEM((NSTEPS, STEP), jnp.int32),
        pltpu.VMEM((5, STEP), jnp.int32),
        pltpu.VMEM_SHARED((DEG_ROWS, DEG_W), jnp.float32),
        pltpu.VMEM_SHARED((CNT_ROWS, 16), jnp.float32),
    ],
)
def _sc_prep(src_hbm, batch_hbm, zdeg_hbm, zcnt_hbm, ones_hbm, ones4_hbm,
             deg_out, cnt_out, ones_v, ones4_v, sidx_v, bidx_v, deg_sh,
             cnt_sh):
    c = lax.axis_index("c")
    s = lax.axis_index("s")
    t = _tile_id()

    @pl.when(s == 0)
    def _():
        pltpu.sync_copy(zdeg_hbm, deg_sh)
        pltpu.sync_copy(zcnt_hbm, cnt_sh)

    pltpu.sync_copy(ones_hbm, ones_v)
    pltpu.sync_copy(ones4_hbm, ones4_v)
    pltpu.sync_copy(src_hbm.at[t], sidx_v)
    pltpu.sync_copy(batch_hbm.at[t], bidx_v)
    plsc.subcore_barrier()

    for i in range(NSTEPS):
        pltpu.sync_copy(ones4_v, deg_sh.at[sidx_v.at[i]], add=True)
    for i in range(5):
        pltpu.sync_copy(ones_v, cnt_sh.at[bidx_v.at[i]], add=True)

    plsc.subcore_barrier()

    @pl.when(s == 0)
    def _():
        pltpu.sync_copy(deg_sh, deg_out.at[c])
        pltpu.sync_copy(cnt_sh, cnt_out.at[c])


_SPLAT_DNUMS = lax.GatherDimensionNumbers(
    offset_dims=(), collapsed_slice_dims=(0,), start_index_map=(0,))


def _splat_lane(vec, k):
    return lax.gather(vec, jnp.full((16, 1), k, jnp.int32), _SPLAT_DNUMS,
                      (1,), mode=lax.GatherScatterMode.PROMISE_IN_BOUNDS)


def _make_edge_kernel(out_ch):
    roww = F_EDGE * out_ch

    @functools.partial(
        pl.kernel,
        out_type=jax.ShapeDtypeStruct((NC, N, out_ch), jnp.float32),
        mesh=_MESH,
        compiler_params=pltpu.CompilerParams(use_tc_tiling_on_sc=False),
        scratch_types=[
            pltpu.VMEM((NSTEPS, STEP), jnp.int32),
            pltpu.VMEM((NSTEPS, STEP), jnp.int32),
            pltpu.VMEM((EPT, 16), jnp.float32),
            pltpu.VMEM((2, STEP, roww), jnp.float32),
            pltpu.VMEM((STEP, out_ch), jnp.float32),
            pltpu.VMEM_SHARED((N, out_ch), jnp.float32),
            pltpu.SemaphoreType.DMA,
            pltpu.SemaphoreType.DMA,
        ],
    )
    def edge_kernel(pn_hbm, srcg_hbm, dst_hbm, ea_hbm, zero_hbm, agg_out,
                    sidx_v, didx_v, ea_v, rows_v, msg_v, agg_sh, sem0, sem1):
        c = lax.axis_index("c")
        s = lax.axis_index("s")
        t = _tile_id()

        @pl.when(s == 0)
        def _():
            pltpu.sync_copy(zero_hbm, agg_sh)

        pltpu.sync_copy(srcg_hbm.at[t], sidx_v)
        pltpu.sync_copy(dst_hbm.at[t], didx_v)
        pltpu.sync_copy(ea_hbm.at[t], ea_v)
        plsc.subcore_barrier()

        sems = (sem0, sem1)
        cps = [None, None]
        cps[0] = pltpu.async_copy(pn_hbm.at[sidx_v.at[0]], rows_v.at[0],
                                  sems[0])
        for step in range(NSTEPS):
            buf = step % 2
            if step + 1 < NSTEPS:
                nbuf = (step + 1) % 2
                cps[nbuf] = pltpu.async_copy(
                    pn_hbm.at[sidx_v.at[step + 1]], rows_v.at[nbuf],
                    sems[nbuf])
            cps[buf].wait()

            def edge_body(e, carry):
                eav = ea_v[step * STEP + e]
                eak = [_splat_lane(eav, k) for k in range(F_EDGE)]
                for j in range(out_ch // 16):
                    acc = eak[0] * rows_v[buf, e, pl.ds(j * 16, 16)]
                    for k in range(1, F_EDGE):
                        acc = acc + eak[k] * rows_v[
                            buf, e, pl.ds(k * out_ch + j * 16, 16)]
                    msg_v[e, pl.ds(j * 16, 16)] = acc
                return carry

            lax.fori_loop(0, STEP, edge_body, 0, unroll=2)
            pltpu.sync_copy(msg_v, agg_sh.at[didx_v.at[step]], add=True)

        plsc.subcore_barrier()

        @pl.when(s == 0)
        def _():
            pltpu.sync_copy(agg_sh, agg_out.at[c])

    return edge_kernel


_edge32 = _make_edge_kernel(32)


@functools.partial(
    pl.kernel,
    out_type=jax.ShapeDtypeStruct((NC, CNT_ROWS, 64), jnp.float32),
    mesh=_MESH,
    compiler_params=pltpu.CompilerParams(use_tc_tiling_on_sc=False),
    scratch_types=[
        pltpu.VMEM((5, STEP), jnp.int32),
        pltpu.VMEM((STEP, 64), jnp.float32),
        pltpu.VMEM_SHARED((CNT_ROWS, 64), jnp.float32),
    ],
)
def _sc_pool(h_hbm, batch_hbm, zpool_hbm, pool_out, bidx_v, rows_v, pool_sh):
    c = lax.axis_index("c")
    s = lax.axis_index("s")
    t = _tile_id()

    @pl.when(s == 0)
    def _():
        pltpu.sync_copy(zpool_hbm, pool_sh)

    pltpu.sync_copy(batch_hbm.at[t], bidx_v)
    plsc.subcore_barrier()

    for i in range(5):
        pltpu.sync_copy(h_hbm.at[pl.ds(t * NPT + i * STEP, STEP)], rows_v)
        pltpu.sync_copy(rows_v, pool_sh.at[bidx_v.at[i]], add=True)

    plsc.subcore_barrier()

    @pl.when(s == 0)
    def _():
        pltpu.sync_copy(pool_sh, pool_out.at[c])


RBLK = 2000
NGRID = N // RBLK


def _norm_blk(dA_blk, dB_blk):
    deg = dA_blk[:, 0:1] + dB_blk[:, 0:1]
    return jnp.where(deg > 0, 1.0 / deg, 0.0)


def _rspec(cols):
    return pl.BlockSpec((RBLK, cols), lambda b: (b, 0))


def _full(shape):
    return pl.BlockSpec(shape, lambda b: tuple(0 for _ in shape))


def _tc_prep1(x, dA, dB, wcat1, wres1t):
    def body(x_ref, dA_ref, dB_ref, wc_ref, wr_ref, pn_ref, res_ref):
        norm = _norm_blk(dA_ref[...], dB_ref[...])
        xn = x_ref[...] * norm
        pn_ref[...] = jnp.dot(xn, wc_ref[...],
                              preferred_element_type=jnp.float32)
        res_ref[...] = jnp.dot(x_ref[...], wr_ref[...],
                               preferred_element_type=jnp.float32)

    return pl.pallas_call(
        body,
        grid=(NGRID,),
        in_specs=[_rspec(F_IN), _rspec(DEG_W), _rspec(DEG_W),
                  _full((F_IN, F_EDGE * 32)), _full((F_IN, 32))],
        out_specs=(_rspec(F_EDGE * 32), _rspec(32)),
        out_shape=(jax.ShapeDtypeStruct((N, F_EDGE * 32), jnp.float32),
                   jax.ShapeDtypeStruct((N, 32), jnp.float32)),
    )(x, dA, dB, wcat1, wres1t)


def _accum_stats(t, b, ssum_ref, ssq_ref):
    s = jnp.sum(t, axis=0, keepdims=True)
    q = jnp.sum(t * t, axis=0, keepdims=True)

    @pl.when(b == 0)
    def _():
        ssum_ref[...] = s
        ssq_ref[...] = q

    @pl.when(b > 0)
    def _():
        ssum_ref[...] += s
        ssq_ref[...] += q


def _tc_stats1(aggs1, res1, b1):
    def body(agg_ref, res_ref, b_ref, t_ref, ssum_ref, ssq_ref):
        t = agg_ref[0] + agg_ref[1] + res_ref[...] + b_ref[...]
        t_ref[...] = t
        _accum_stats(t, pl.program_id(0), ssum_ref, ssq_ref)

    st_spec = pl.BlockSpec((1, 32), lambda b: (0, 0))
    return pl.pallas_call(
        body,
        grid=(NGRID,),
        in_specs=[pl.BlockSpec((2, RBLK, 32), lambda b: (0, b, 0)),
                  _rspec(32), _full((1, 32))],
        out_specs=(_rspec(32), st_spec, st_spec),
        out_shape=(jax.ShapeDtypeStruct((N, 32), jnp.float32),
                   jax.ShapeDtypeStruct((1, 32), jnp.float32),
                   jax.ShapeDtypeStruct((1, 32), jnp.float32)),
    )(aggs1, res1, b1)


def _tc_stats2(aggs2a, aggs2b, res2, b2):
    def body(agga_ref, aggb_ref, res_ref, b_ref, t_ref, ssum_ref, ssq_ref):
        agg = jnp.concatenate(
            [agga_ref[0] + agga_ref[1], aggb_ref[0] + aggb_ref[1]], axis=1)
        t = agg + res_ref[...] + b_ref[...]
        t_ref[...] = t
        _accum_stats(t, pl.program_id(0), ssum_ref, ssq_ref)

    st_spec = pl.BlockSpec((1, 64), lambda b: (0, 0))
    return pl.pallas_call(
        body,
        grid=(NGRID,),
        in_specs=[pl.BlockSpec((2, RBLK, 32), lambda b: (0, b, 0)),
                  pl.BlockSpec((2, RBLK, 32), lambda b: (0, b, 0)),
                  _rspec(64), _full((1, 64))],
        out_specs=(_rspec(64), st_spec, st_spec),
        out_shape=(jax.ShapeDtypeStruct((N, 64), jnp.float32),
                   jax.ShapeDtypeStruct((1, 64), jnp.float32),
                   jax.ShapeDtypeStruct((1, 64), jnp.float32)),
    )(aggs2a, aggs2b, res2, b2)


def _bn_relu_from_stats(t, ssum, ssq, g, be):
    mu = ssum * (1.0 / N)
    var = ssq * (1.0 / N) - mu * mu
    return jnp.maximum(g * (t - mu) / jnp.sqrt(var + 1e-5) + be, 0.0)


def _tc_apply1(t1, ssum1, ssq1, g1, be1, dA, dB, wcat2a, wcat2b, wres2t):
    def body(t_ref, ssum_ref, ssq_ref, g_ref, be_ref, dA_ref, dB_ref,
             wca_ref, wcb_ref, wr_ref, pna_ref, pnb_ref, res2_ref):
        h = _bn_relu_from_stats(t_ref[...], ssum_ref[...], ssq_ref[...],
                                g_ref[...], be_ref[...])
        norm = _norm_blk(dA_ref[...], dB_ref[...])
        hn = h * norm
        pna_ref[...] = jnp.dot(hn, wca_ref[...],
                               preferred_element_type=jnp.float32)
        pnb_ref[...] = jnp.dot(hn, wcb_ref[...],
                               preferred_element_type=jnp.float32)
        res2_ref[...] = jnp.dot(h, wr_ref[...],
                                preferred_element_type=jnp.float32)

    return pl.pallas_call(
        body,
        grid=(NGRID,),
        in_specs=[_rspec(32), _full((1, 32)), _full((1, 32)),
                  _full((1, 32)), _full((1, 32)),
                  _rspec(DEG_W), _rspec(DEG_W),
                  _full((32, F_EDGE * 32)), _full((32, F_EDGE * 32)),
                  _full((32, 64))],
        out_specs=(_rspec(F_EDGE * 32), _rspec(F_EDGE * 32), _rspec(64)),
        out_shape=(jax.ShapeDtypeStruct((N, F_EDGE * 32), jnp.float32),
                   jax.ShapeDtypeStruct((N, F_EDGE * 32), jnp.float32),
                   jax.ShapeDtypeStruct((N, 64), jnp.float32)),
    )(t1, ssum1, ssq1, g1, be1, dA, dB, wcat2a, wcat2b, wres2t)


PBLK = 2048
NPGRID = N_PAD // PBLK


def _tc_apply2(t2, ssum2, ssq2, g2, be2):
    def body(t_ref, ssum_ref, ssq_ref, g_ref, be_ref, h_ref):
        h = _bn_relu_from_stats(t_ref[...], ssum_ref[...], ssq_ref[...],
                                g_ref[...], be_ref[...])
        row0 = pl.program_id(0) * PBLK
        rows = row0 + lax.broadcasted_iota(jnp.int32, (PBLK, 64), 0)
        h_ref[...] = jnp.where(rows < N, h, 0.0)

    return pl.pallas_call(
        body,
        grid=(NPGRID,),
        in_specs=[pl.BlockSpec((PBLK, 64), lambda b: (b, 0)),
                  _full((1, 64)), _full((1, 64)), _full((1, 64)),
                  _full((1, 64))],
        out_specs=pl.BlockSpec((PBLK, 64), lambda b: (b, 0)),
        out_shape=jax.ShapeDtypeStruct((N_PAD, 64), jnp.float32),
    )(t2, ssum2, ssq2, g2, be2)


def _tc_head(pools, cA, cB, wfc1t, bfc1, wfc2t, bfc2):
    def body(pool_ref, cA_ref, cB_ref, w1_ref, b1_ref, w2_ref, b2_ref,
             mu_ref, ls_ref):
        sums = pool_ref[0, 0:B, :] + pool_ref[1, 0:B, :]
        cnt = cA_ref[0:B, 0:1] + cB_ref[0:B, 0:1]
        pooled = sums / jnp.maximum(cnt, 1.0)
        z1 = jnp.maximum(
            jnp.dot(pooled, w1_ref[...], preferred_element_type=jnp.float32)
            + b1_ref[...], 0.0)
        z = jnp.dot(z1, w2_ref[...],
                    preferred_element_type=jnp.float32) + b2_ref[...]
        mu_ref[...] = z[:, 0:LATENT]
        ls_ref[...] = z[:, LATENT:2 * LATENT]

    return pl.pallas_call(
        body,
        out_shape=(jax.ShapeDtypeStruct((B, LATENT), jnp.float32),
                   jax.ShapeDtypeStruct((B, LATENT), jnp.float32)),
    )(pools, cA, cB, wfc1t, bfc1, wfc2t, bfc2)


def _wcat(w_edge, out_ch, in_ch):
    return (w_edge.reshape(out_ch, in_ch, F_EDGE)
            .transpose(1, 2, 0).reshape(in_ch, F_EDGE * out_ch))


def kernel(x, edge_index, edge_attr, batch, W_edge1, b1, W_res1, g1, be1,
           W_edge2, b2, W_res2, g2, be2, W_fc1, b_fc1, W_fc2, b_fc2):
    src = edge_index[0]
    dst = edge_index[1]
    epad = E_PAD - E
    espread = jnp.arange(epad, dtype=jnp.int32)
    srcg = jnp.concatenate([src, espread % N])
    srcg = srcg.reshape(NW, NSTEPS, STEP)
    srcd = jnp.concatenate([src, N + espread % (DEG_ROWS - N)])
    srcd = srcd.reshape(NW, NSTEPS, STEP)
    dstp = jnp.concatenate([dst, espread % N])
    dstp = dstp.reshape(NW, NSTEPS, STEP)
    eap = jnp.pad(edge_attr, ((0, epad), (0, 16 - F_EDGE)))
    eap = eap.reshape(NW, EPT, 16)
    nspread = jnp.arange(N_PAD - N, dtype=jnp.int32)
    batchp = jnp.concatenate([batch, B + nspread % (CNT_ROWS - B)])
    batchp = batchp.reshape(NW, 5, STEP)

    zdeg = jnp.zeros((DEG_ROWS, DEG_W), jnp.float32)
    zcnt = jnp.zeros((CNT_ROWS, 16), jnp.float32)
    z32 = jnp.zeros((N, 32), jnp.float32)
    zpool = jnp.zeros((CNT_ROWS, 64), jnp.float32)
    ones16 = jnp.ones((STEP, 16), jnp.float32)
    ones4 = jnp.ones((STEP, DEG_W), jnp.float32)

    wcat1 = _wcat(W_edge1, 32, F_IN)
    wcat2 = _wcat(W_edge2, 64, 32).reshape(32, F_EDGE, 64)
    wcat2a = wcat2[:, :, :32].reshape(32, F_EDGE * 32)
    wcat2b = wcat2[:, :, 32:].reshape(32, F_EDGE * 32)

    degs, cnts = _sc_prep(srcd, batchp, zdeg, zcnt, ones16, ones4)
    dA, dB = degs[0], degs[1]

    pn1, res1 = _tc_prep1(x, dA, dB, wcat1, W_res1.T)
    aggs1 = _edge32(pn1, srcg, dstp, eap, z32)
    t1, ssum1, ssq1 = _tc_stats1(aggs1, res1, b1.reshape(1, 32))
    pn2a, pn2b, res2 = _tc_apply1(t1, ssum1, ssq1, g1.reshape(1, 32),
                                  be1.reshape(1, 32), dA, dB,
                                  wcat2a, wcat2b, W_res2.T)
    aggs2a = _edge32(pn2a, srcg, dstp, eap, z32)
    aggs2b = _edge32(pn2b, srcg, dstp, eap, z32)
    t2, ssum2, ssq2 = _tc_stats2(aggs2a, aggs2b, res2, b2.reshape(1, 64))
    h2 = _tc_apply2(t2, ssum2, ssq2, g2.reshape(1, 64), be2.reshape(1, 64))
    pools = _sc_pool(h2, batchp, zpool)
    mu, log_sigma = _tc_head(pools, cnts[0], cnts[1], W_fc1.T,
                             b_fc1.reshape(1, -1), W_fc2.T,
                             b_fc2.reshape(1, -1))
    return (mu, log_sigma)


# speedup vs baseline: 1.5962x; 1.0031x over previous
import functools

import jax
import jax.numpy as jnp
from jax import lax
from jax.experimental import pallas as pl
from jax.experimental.pallas import tpu as pltpu
from jax.experimental.pallas import tpu_sc as plsc

N = 20000
E = 80000
F_IN = 16
F_EDGE = 4
B = 512
LATENT = 64

NC = 2
NS = 16
NW = NC * NS

STEP = 128
NSTEPS = 20
EPT = STEP * NSTEPS
E_PAD = NW * EPT

NPT = 640
N_PAD = NW * NPT
CNT_ROWS = 640
DEG_ROWS = N_PAD
DEG_W = 16

_MESH = plsc.VectorSubcoreMesh(core_axis_name="c", subcore_axis_name="s")


def _tile_id():
    return lax.axis_index("c") * NS + lax.axis_index("s")


@functools.partial(
    pl.kernel,
    out_type=(
        jax.ShapeDtypeStruct((NC, DEG_ROWS, DEG_W), jnp.float32),
        jax.ShapeDtypeStruct((NC, CNT_ROWS, 16), jnp.float32),
    ),
    mesh=_MESH,
    compiler_params=pltpu.CompilerParams(use_tc_tiling_on_sc=False),
    scratch_types=[
        pltpu.VMEM((STEP, 16), jnp.float32),
        pltpu.VMEM((STEP, DEG_W), jnp.float32),
        pltpu.VMEM((NSTEPS, STEP), jnp.int32),
        pltpu.VMEM((5, STEP), jnp.int32),
        pltpu.VMEM_SHARED((DEG_ROWS, DEG_W), jnp.float32),
        pltpu.VMEM_SHARED((CNT_ROWS, 16), jnp.float32),
    ],
)
def _sc_prep(src_hbm, batch_hbm, zdeg_hbm, zcnt_hbm, ones_hbm, ones4_hbm,
             deg_out, cnt_out, ones_v, ones4_v, sidx_v, bidx_v, deg_sh,
             cnt_sh):
    c = lax.axis_index("c")
    s = lax.axis_index("s")
    t = _tile_id()

    @pl.when(s == 0)
    def _():
        pltpu.sync_copy(zdeg_hbm, deg_sh)
        pltpu.sync_copy(zcnt_hbm, cnt_sh)

    pltpu.sync_copy(ones_hbm, ones_v)
    pltpu.sync_copy(ones4_hbm, ones4_v)
    pltpu.sync_copy(src_hbm.at[t], sidx_v)
    pltpu.sync_copy(batch_hbm.at[t], bidx_v)
    plsc.subcore_barrier()

    for i in range(NSTEPS):
        pltpu.sync_copy(ones4_v, deg_sh.at[sidx_v.at[i]], add=True)
    for i in range(5):
        pltpu.sync_copy(ones_v, cnt_sh.at[bidx_v.at[i]], add=True)

    plsc.subcore_barrier()

    @pl.when(s == 0)
    def _():
        pltpu.sync_copy(deg_sh, deg_out.at[c])
        pltpu.sync_copy(cnt_sh, cnt_out.at[c])


_SPLAT_DNUMS = lax.GatherDimensionNumbers(
    offset_dims=(), collapsed_slice_dims=(0,), start_index_map=(0,))


def _splat_lane(vec, k):
    return lax.gather(vec, jnp.full((16, 1), k, jnp.int32), _SPLAT_DNUMS,
                      (1,), mode=lax.GatherScatterMode.PROMISE_IN_BOUNDS)


def _make_edge_kernel(out_ch):
    roww = F_EDGE * out_ch
    R15 = 1248
    RLAST = N - 15 * R15

    @functools.partial(
        pl.kernel,
        out_type=jax.ShapeDtypeStruct((NC, N, out_ch), jnp.float32),
        mesh=_MESH,
        compiler_params=pltpu.CompilerParams(use_tc_tiling_on_sc=False),
        scratch_types=[
            pltpu.VMEM((NSTEPS, STEP), jnp.int32),
            pltpu.VMEM((NSTEPS, STEP), jnp.int32),
            pltpu.VMEM((EPT, 16), jnp.float32),
            pltpu.VMEM((2, STEP, roww), jnp.float32),
            pltpu.VMEM((STEP, out_ch), jnp.float32),
            pltpu.VMEM_SHARED((N, out_ch), jnp.float32),
            pltpu.SemaphoreType.DMA,
            pltpu.SemaphoreType.DMA,
        ],
    )
    def edge_kernel(pn_hbm, srcg_hbm, dst_hbm, ea_hbm, zero_hbm, agg_out,
                    sidx_v, didx_v, ea_v, rows_v, msg_v, agg_sh, sem0, sem1):
        c = lax.axis_index("c")
        s = lax.axis_index("s")
        t = _tile_id()

        @pl.when(s < 15)
        def _():
            sl = pl.ds(s * R15, R15)
            pltpu.sync_copy(zero_hbm.at[sl], agg_sh.at[sl])

        @pl.when(s == 15)
        def _():
            sl = pl.ds(15 * R15, RLAST)
            pltpu.sync_copy(zero_hbm.at[sl], agg_sh.at[sl])

        pltpu.sync_copy(srcg_hbm.at[t], sidx_v)
        pltpu.sync_copy(dst_hbm.at[t], didx_v)
        pltpu.sync_copy(ea_hbm.at[t], ea_v)
        plsc.subcore_barrier()

        sems = (sem0, sem1)
        cps = [None, None]
        cps[0] = pltpu.async_copy(pn_hbm.at[sidx_v.at[0]], rows_v.at[0],
                                  sems[0])
        for step in range(NSTEPS):
            buf = step % 2
            if step + 1 < NSTEPS:
                nbuf = (step + 1) % 2
                cps[nbuf] = pltpu.async_copy(
                    pn_hbm.at[sidx_v.at[step + 1]], rows_v.at[nbuf],
                    sems[nbuf])
            cps[buf].wait()

            def edge_body(e, carry):
                eav = ea_v[step * STEP + e]
                eak = [_splat_lane(eav, k) for k in range(F_EDGE)]
                for j in range(out_ch // 16):
                    acc = eak[0] * rows_v[buf, e, pl.ds(j * 16, 16)]
                    for k in range(1, F_EDGE):
                        acc = acc + eak[k] * rows_v[
                            buf, e, pl.ds(k * out_ch + j * 16, 16)]
                    msg_v[e, pl.ds(j * 16, 16)] = acc
                return carry

            lax.fori_loop(0, STEP, edge_body, 0)
            pltpu.sync_copy(msg_v, agg_sh.at[didx_v.at[step]], add=True)

        plsc.subcore_barrier()

        @pl.when(s < 15)
        def _():
            sl = pl.ds(s * R15, R15)
            pltpu.sync_copy(agg_sh.at[sl], agg_out.at[c].at[sl])

        @pl.when(s == 15)
        def _():
            sl = pl.ds(15 * R15, RLAST)
            pltpu.sync_copy(agg_sh.at[sl], agg_out.at[c].at[sl])

    return edge_kernel


_edge32 = _make_edge_kernel(32)


@functools.partial(
    pl.kernel,
    out_type=jax.ShapeDtypeStruct((NC, CNT_ROWS, 64), jnp.float32),
    mesh=_MESH,
    compiler_params=pltpu.CompilerParams(use_tc_tiling_on_sc=False),
    scratch_types=[
        pltpu.VMEM((5, STEP), jnp.int32),
        pltpu.VMEM((STEP, 64), jnp.float32),
        pltpu.VMEM_SHARED((CNT_ROWS, 64), jnp.float32),
    ],
)
def _sc_pool(h_hbm, batch_hbm, zpool_hbm, pool_out, bidx_v, rows_v, pool_sh):
    c = lax.axis_index("c")
    s = lax.axis_index("s")
    t = _tile_id()

    @pl.when(s == 0)
    def _():
        pltpu.sync_copy(zpool_hbm, pool_sh)

    pltpu.sync_copy(batch_hbm.at[t], bidx_v)
    plsc.subcore_barrier()

    for i in range(5):
        pltpu.sync_copy(h_hbm.at[pl.ds(t * NPT + i * STEP, STEP)], rows_v)
        pltpu.sync_copy(rows_v, pool_sh.at[bidx_v.at[i]], add=True)

    plsc.subcore_barrier()

    @pl.when(s == 0)
    def _():
        pltpu.sync_copy(pool_sh, pool_out.at[c])


RBLK = 2000
NGRID = N // RBLK


def _norm_blk(dA_blk, dB_blk):
    deg = dA_blk[:, 0:1] + dB_blk[:, 0:1]
    return jnp.where(deg > 0, 1.0 / deg, 0.0)


def _rspec(cols):
    return pl.BlockSpec((RBLK, cols), lambda b: (b, 0))


def _full(shape):
    return pl.BlockSpec(shape, lambda b: tuple(0 for _ in shape))


def _tc_prep1(x, dA, dB, wcat1, wres1t):
    def body(x_ref, dA_ref, dB_ref, wc_ref, wr_ref, pn_ref, res_ref):
        norm = _norm_blk(dA_ref[...], dB_ref[...])
        xn = x_ref[...] * norm
        pn_ref[...] = jnp.dot(xn, wc_ref[...],
                              preferred_element_type=jnp.float32)
        res_ref[...] = jnp.dot(x_ref[...], wr_ref[...],
                               preferred_element_type=jnp.float32)

    return pl.pallas_call(
        body,
        grid=(NGRID,),
        in_specs=[_rspec(F_IN), _rspec(DEG_W), _rspec(DEG_W),
                  _full((F_IN, F_EDGE * 32)), _full((F_IN, 32))],
        out_specs=(_rspec(F_EDGE * 32), _rspec(32)),
        out_shape=(jax.ShapeDtypeStruct((N, F_EDGE * 32), jnp.float32),
                   jax.ShapeDtypeStruct((N, 32), jnp.float32)),
    )(x, dA, dB, wcat1, wres1t)


def _accum_stats(t, b, ssum_ref, ssq_ref):
    s = jnp.sum(t, axis=0, keepdims=True)
    q = jnp.sum(t * t, axis=0, keepdims=True)

    @pl.when(b == 0)
    def _():
        ssum_ref[...] = s
        ssq_ref[...] = q

    @pl.when(b > 0)
    def _():
        ssum_ref[...] += s
        ssq_ref[...] += q


def _tc_stats1(aggs1, res1, b1):
    def body(agg_ref, res_ref, b_ref, t_ref, ssum_ref, ssq_ref):
        t = agg_ref[0] + agg_ref[1] + res_ref[...] + b_ref[...]
        t_ref[...] = t
        _accum_stats(t, pl.program_id(0), ssum_ref, ssq_ref)

    st_spec = pl.BlockSpec((1, 32), lambda b: (0, 0))
    return pl.pallas_call(
        body,
        grid=(NGRID,),
        in_specs=[pl.BlockSpec((2, RBLK, 32), lambda b: (0, b, 0)),
                  _rspec(32), _full((1, 32))],
        out_specs=(_rspec(32), st_spec, st_spec),
        out_shape=(jax.ShapeDtypeStruct((N, 32), jnp.float32),
                   jax.ShapeDtypeStruct((1, 32), jnp.float32),
                   jax.ShapeDtypeStruct((1, 32), jnp.float32)),
    )(aggs1, res1, b1)


def _tc_stats2(aggs2a, aggs2b, res2, b2):
    def body(agga_ref, aggb_ref, res_ref, b_ref, t_ref, ssum_ref, ssq_ref):
        agg = jnp.concatenate(
            [agga_ref[0] + agga_ref[1], aggb_ref[0] + aggb_ref[1]], axis=1)
        t = agg + res_ref[...] + b_ref[...]
        t_ref[...] = t
        _accum_stats(t, pl.program_id(0), ssum_ref, ssq_ref)

    st_spec = pl.BlockSpec((1, 64), lambda b: (0, 0))
    return pl.pallas_call(
        body,
        grid=(NGRID,),
        in_specs=[pl.BlockSpec((2, RBLK, 32), lambda b: (0, b, 0)),
                  pl.BlockSpec((2, RBLK, 32), lambda b: (0, b, 0)),
                  _rspec(64), _full((1, 64))],
        out_specs=(_rspec(64), st_spec, st_spec),
        out_shape=(jax.ShapeDtypeStruct((N, 64), jnp.float32),
                   jax.ShapeDtypeStruct((1, 64), jnp.float32),
                   jax.ShapeDtypeStruct((1, 64), jnp.float32)),
    )(aggs2a, aggs2b, res2, b2)


def _bn_relu_from_stats(t, ssum, ssq, g, be):
    mu = ssum * (1.0 / N)
    var = ssq * (1.0 / N) - mu * mu
    return jnp.maximum(g * (t - mu) / jnp.sqrt(var + 1e-5) + be, 0.0)


def _tc_apply1(t1, ssum1, ssq1, g1, be1, dA, dB, wcat2a, wcat2b, wres2t):
    def body(t_ref, ssum_ref, ssq_ref, g_ref, be_ref, dA_ref, dB_ref,
             wca_ref, wcb_ref, wr_ref, pna_ref, pnb_ref, res2_ref):
        h = _bn_relu_from_stats(t_ref[...], ssum_ref[...], ssq_ref[...],
                                g_ref[...], be_ref[...])
        norm = _norm_blk(dA_ref[...], dB_ref[...])
        hn = h * norm
        pna_ref[...] = jnp.dot(hn, wca_ref[...],
                               preferred_element_type=jnp.float32)
        pnb_ref[...] = jnp.dot(hn, wcb_ref[...],
                               preferred_element_type=jnp.float32)
        res2_ref[...] = jnp.dot(h, wr_ref[...],
                                preferred_element_type=jnp.float32)

    return pl.pallas_call(
        body,
        grid=(NGRID,),
        in_specs=[_rspec(32), _full((1, 32)), _full((1, 32)),
                  _full((1, 32)), _full((1, 32)),
                  _rspec(DEG_W), _rspec(DEG_W),
                  _full((32, F_EDGE * 32)), _full((32, F_EDGE * 32)),
                  _full((32, 64))],
        out_specs=(_rspec(F_EDGE * 32), _rspec(F_EDGE * 32), _rspec(64)),
        out_shape=(jax.ShapeDtypeStruct((N, F_EDGE * 32), jnp.float32),
                   jax.ShapeDtypeStruct((N, F_EDGE * 32), jnp.float32),
                   jax.ShapeDtypeStruct((N, 64), jnp.float32)),
    )(t1, ssum1, ssq1, g1, be1, dA, dB, wcat2a, wcat2b, wres2t)


PBLK = 2048
NPGRID = N_PAD // PBLK


def _tc_apply2(t2, ssum2, ssq2, g2, be2):
    def body(t_ref, ssum_ref, ssq_ref, g_ref, be_ref, h_ref):
        h = _bn_relu_from_stats(t_ref[...], ssum_ref[...], ssq_ref[...],
                                g_ref[...], be_ref[...])
        row0 = pl.program_id(0) * PBLK
        rows = row0 + lax.broadcasted_iota(jnp.int32, (PBLK, 64), 0)
        h_ref[...] = jnp.where(rows < N, h, 0.0)

    return pl.pallas_call(
        body,
        grid=(NPGRID,),
        in_specs=[pl.BlockSpec((PBLK, 64), lambda b: (b, 0)),
                  _full((1, 64)), _full((1, 64)), _full((1, 64)),
                  _full((1, 64))],
        out_specs=pl.BlockSpec((PBLK, 64), lambda b: (b, 0)),
        out_shape=jax.ShapeDtypeStruct((N_PAD, 64), jnp.float32),
    )(t2, ssum2, ssq2, g2, be2)


def _tc_head(pools, cA, cB, wfc1t, bfc1, wfc2t, bfc2):
    def body(pool_ref, cA_ref, cB_ref, w1_ref, b1_ref, w2_ref, b2_ref,
             mu_ref, ls_ref):
        sums = pool_ref[0, 0:B, :] + pool_ref[1, 0:B, :]
        cnt = cA_ref[0:B, 0:1] + cB_ref[0:B, 0:1]
        pooled = sums / jnp.maximum(cnt, 1.0)
        z1 = jnp.maximum(
            jnp.dot(pooled, w1_ref[...], preferred_element_type=jnp.float32)
            + b1_ref[...], 0.0)
        z = jnp.dot(z1, w2_ref[...],
                    preferred_element_type=jnp.float32) + b2_ref[...]
        mu_ref[...] = z[:, 0:LATENT]
        ls_ref[...] = z[:, LATENT:2 * LATENT]

    return pl.pallas_call(
        body,
        out_shape=(jax.ShapeDtypeStruct((B, LATENT), jnp.float32),
                   jax.ShapeDtypeStruct((B, LATENT), jnp.float32)),
    )(pools, cA, cB, wfc1t, bfc1, wfc2t, bfc2)


def _wcat(w_edge, out_ch, in_ch):
    return (w_edge.reshape(out_ch, in_ch, F_EDGE)
            .transpose(1, 2, 0).reshape(in_ch, F_EDGE * out_ch))


def kernel(x, edge_index, edge_attr, batch, W_edge1, b1, W_res1, g1, be1,
           W_edge2, b2, W_res2, g2, be2, W_fc1, b_fc1, W_fc2, b_fc2):
    src = edge_index[0]
    dst = edge_index[1]
    epad = E_PAD - E
    espread = jnp.arange(epad, dtype=jnp.int32)
    srcg = jnp.concatenate([src, espread % N])
    srcg = srcg.reshape(NW, NSTEPS, STEP)
    srcd = jnp.concatenate([src, N + espread % (DEG_ROWS - N)])
    srcd = srcd.reshape(NW, NSTEPS, STEP)
    dstp = jnp.concatenate([dst, espread % N])
    dstp = dstp.reshape(NW, NSTEPS, STEP)
    eap = jnp.pad(edge_attr, ((0, epad), (0, 16 - F_EDGE)))
    eap = eap.reshape(NW, EPT, 16)
    nspread = jnp.arange(N_PAD - N, dtype=jnp.int32)
    batchp = jnp.concatenate([batch, B + nspread % (CNT_ROWS - B)])
    batchp = batchp.reshape(NW, 5, STEP)

    zdeg = jnp.zeros((DEG_ROWS, DEG_W), jnp.float32)
    zcnt = jnp.zeros((CNT_ROWS, 16), jnp.float32)
    z32 = jnp.zeros((N, 32), jnp.float32)
    zpool = jnp.zeros((CNT_ROWS, 64), jnp.float32)
    ones16 = jnp.ones((STEP, 16), jnp.float32)
    ones4 = jnp.ones((STEP, DEG_W), jnp.float32)

    wcat1 = _wcat(W_edge1, 32, F_IN)
    wcat2 = _wcat(W_edge2, 64, 32).reshape(32, F_EDGE, 64)
    wcat2a = wcat2[:, :, :32].reshape(32, F_EDGE * 32)
    wcat2b = wcat2[:, :, 32:].reshape(32, F_EDGE * 32)

    degs, cnts = _sc_prep(srcd, batchp, zdeg, zcnt, ones16, ones4)
    dA, dB = degs[0], degs[1]

    pn1, res1 = _tc_prep1(x, dA, dB, wcat1, W_res1.T)
    aggs1 = _edge32(pn1, srcg, dstp, eap, z32)
    t1, ssum1, ssq1 = _tc_stats1(aggs1, res1, b1.reshape(1, 32))
    pn2a, pn2b, res2 = _tc_apply1(t1, ssum1, ssq1, g1.reshape(1, 32),
                                  be1.reshape(1, 32), dA, dB,
                                  wcat2a, wcat2b, W_res2.T)
    aggs2a = _edge32(pn2a, srcg, dstp, eap, z32)
    aggs2b = _edge32(pn2b, srcg, dstp, eap, z32)
    t2, ssum2, ssq2 = _tc_stats2(aggs2a, aggs2b, res2, b2.reshape(1, 64))
    h2 = _tc_apply2(t2, ssum2, ssq2, g2.reshape(1, 64), be2.reshape(1, 64))
    pools = _sc_pool(h2, batchp, zpool)
    mu, log_sigma = _tc_head(pools, cnts[0], cnts[1], W_fc1.T,
                             b_fc1.reshape(1, -1), W_fc2.T,
                             b_fc2.reshape(1, -1))
    return (mu, log_sigma)
